# baseline (speedup 1.0000x reference)
import functools

import jax
import jax.numpy as jnp
from jax.experimental import pallas as pl
from jax.experimental.pallas import tpu as pltpu
from jax.experimental.pallas import tpu_sc as plsc

_RHO = 0.2
_TAU = 1.0
_TBLK = 2048
_T_SC = 2048
_NW = 32
_CH = 64


def _sc_matvec(emb_flat, w_flat, *, B, T, D, T_tc):
    wpb = _NW // B
    tpw = _T_SC // wpb
    nch = tpw // _CH
    cw = _CH * D
    mesh = plsc.VectorSubcoreMesh(core_axis_name="c", subcore_axis_name="s")

    @functools.partial(
        pl.kernel, mesh=mesh,
        out_type=jax.ShapeDtypeStruct((B * _T_SC,), jnp.float32),
        scratch_types=[
            pltpu.VMEM((cw,), jnp.float32),
            pltpu.VMEM((cw,), jnp.float32),
            pltpu.VMEM((D,), jnp.float32),
            pltpu.VMEM((tpw,), jnp.float32),
            pltpu.SemaphoreType.DMA,
            pltpu.SemaphoreType.DMA,
        ])
    def k(emb_hbm, w_hbm, out_hbm, buf0, buf1, wv, outv, sem0, sem1):
        c = jax.lax.axis_index("c")
        s = jax.lax.axis_index("s")
        wid = s * 2 + c
        brow = wid // wpb
        sub = wid % wpb
        src0 = (brow * T + T_tc + sub * tpw) * D

        pltpu.sync_copy(w_hbm, wv)
        for j in range(D // 16):
            u = jax.lax.bitcast_convert_type(wv[pl.ds(j * 16, 16)], jnp.int32)
            u = (u + jnp.int32(0x7FFF) +
                 (jax.lax.shift_right_logical(u, 16) & 1)) & jnp.int32(-65536)
            wv[pl.ds(j * 16, 16)] = jax.lax.bitcast_convert_type(u, jnp.float32)

        bufs = (buf0, buf1)
        sems = (sem0, sem1)

        def start(i):
            pltpu.make_async_copy(
                emb_hbm.at[pl.ds(src0 + i * cw, cw)], bufs[i % 2],
                sems[i % 2]).start()

        def wait(i):
            pltpu.make_async_copy(
                emb_hbm.at[pl.ds(0, cw)], bufs[i % 2], sems[i % 2]).wait()

        start(0)
        start(1)
        for i in range(nch):
            wait(i)
            buf = bufs[i % 2]

            lane = jax.lax.iota(jnp.int32, 16)
            perms = [lane ^ sh for sh in (8, 4, 2, 1)]

            def row_tile(rt, _):
                base = rt * 16 * D

                def row(r, res):
                    acc = jnp.zeros((16,), jnp.float32)
                    for j in range(D // 16):
                        e = buf[pl.ds(base + r * D + j * 16, 16)]
                        u = jax.lax.bitcast_convert_type(e, jnp.int32)
                        u = (u + jnp.int32(0x7FFF) +
                             (jax.lax.shift_right_logical(u, 16) & 1)
                             ) & jnp.int32(-65536)
                        e = jax.lax.bitcast_convert_type(u, jnp.float32)
                        acc = acc + e * wv[pl.ds(j * 16, 16)]
                    for p in perms:
                        acc = acc + acc.at[p].get(mode="promise_in_bounds")
                    return jnp.where(lane == r, acc, res)

                res = jax.lax.fori_loop(0, 16, row,
                                        jnp.zeros((16,), jnp.float32))
                outv[pl.ds(i * _CH + rt * 16, 16)] = res
                return 0

            jax.lax.fori_loop(0, _CH // 16, row_tile, 0)
            if i + 2 < nch:
                start(i + 2)
        pltpu.sync_copy(outv, out_hbm.at[pl.ds(wid * tpw, tpw)])

    return k(emb_flat, w_flat)


def _tc_stream_kernel(emb_ref, w_ref, scores_ref):
    t = pl.program_id(1)
    emb = emb_ref[0]
    w = w_ref[...]
    scores_ref[0, :, pl.ds(t * _TBLK, _TBLK)] = jax.lax.dot_general(
        w, emb, (((1,), (1,)), ((), ())),
        preferred_element_type=jnp.float32)


def _tail_kernel(raw_tc_ref, raw_sc_ref, attn_ref, b_ref, g_ref, z_ref,
                 *, nB, T):
    attn = attn_ref[...]
    bias = b_ref[0, 0]
    raw = jnp.concatenate([raw_tc_ref[...], raw_sc_ref[...]], axis=1)
    s = attn * raw + bias
    s = jnp.where(attn == 0.0, jnp.float32(-1e9), s)
    t_eff = jnp.sum(attn, axis=1, keepdims=True)
    kf = jnp.maximum(jnp.round(jnp.float32(_RHO) * t_eff), 1.0)
    s = s / jnp.float32(_TAU)
    e = jnp.exp(s - jnp.max(s, axis=1, keepdims=True))
    z = kf * (e / jnp.sum(e, axis=1, keepdims=True))

    bits = jax.lax.bitcast_convert_type(z, jnp.int32)
    key = jnp.where(bits < 0, bits ^ jnp.int32(0x7FFFFFFF), bits)
    ub = jax.lax.bitcast_convert_type(key, jnp.uint32) ^ jnp.uint32(0x80000000)
    ki = kf.astype(jnp.int32)

    def _cnt_ge(c):
        return jnp.sum((ub >= c).astype(jnp.int32), axis=1, keepdims=True)

    def sbody(i, v):
        p_hi = (jnp.int32(15) - i) * 2 + 1
        b_hi = jax.lax.shift_left(jnp.uint32(1), p_hi.astype(jnp.uint32))
        b_lo = jax.lax.shift_left(jnp.uint32(1), (p_hi - 1).astype(jnp.uint32))
        c01 = v | b_lo
        c10 = v | b_hi
        c11 = c10 | b_lo
        n01, n10, n11 = _cnt_ge(c01), _cnt_ge(c10), _cnt_ge(c11)
        return jnp.where(n11 >= ki, c11,
                         jnp.where(n10 >= ki, c10,
                                   jnp.where(n01 >= ki, c01, v)))

    v = jax.lax.fori_loop(0, 16, sbody, jnp.zeros((nB, 1), jnp.uint32))

    gt = ub > v
    eq = ub == v
    need = ki - jnp.sum(gt.astype(jnp.int32), axis=1, keepdims=True)
    c_eq = jnp.sum(eq.astype(jnp.int32), axis=1, keepdims=True)
    idx = jax.lax.broadcasted_iota(jnp.int32, (1, T), 1)

    ties = jnp.logical_not(jnp.all(c_eq == need))

    def ibody(i, lohi):
        lo, hi = lohi
        mid = (lo + hi) // 2
        cnt = jnp.sum((eq & (idx <= mid)).astype(jnp.int32),
                      axis=1, keepdims=True)
        take = cnt >= need
        return (jnp.where(take, lo, mid + 1), jnp.where(take, mid, hi))

    lo0 = jnp.where(ties, 0, T - 1) * jnp.ones((nB, 1), jnp.int32)
    hi0 = jnp.full((nB, 1), T - 1, jnp.int32)
    trip = jnp.where(ties, 13, 0)
    lo, _ = jax.lax.fori_loop(0, trip, ibody, (lo0, hi0))
    h = (gt | (eq & (idx <= lo) & (need > 0))).astype(jnp.float32)
    g_ref[...] = h
    z_ref[...] = z


def kernel(embeddings, attn, W, b):
    B, T, D = embeddings.shape
    T_tc = T - _T_SC
    nT = T_tc // _TBLK
    b2 = b.reshape(1, 1)

    w_sc = W.astype(jnp.bfloat16).astype(jnp.float32).reshape(-1)
    raw_sc = _sc_matvec(embeddings.reshape(-1), w_sc,
                        B=B, T=T, D=D, T_tc=T_tc)

    raw_tc = pl.pallas_call(
        _tc_stream_kernel,
        grid=(B, nT),
        in_specs=[
            pl.BlockSpec((1, _TBLK, D), lambda bb, tt: (bb, tt, 0)),
            pl.BlockSpec((1, D), lambda bb, tt: (0, 0)),
        ],
        out_specs=pl.BlockSpec((1, 1, T_tc), lambda bb, tt: (bb, 0, 0)),
        out_shape=jax.ShapeDtypeStruct((B, 1, T_tc), jnp.float32),
    )(embeddings, W)
    raw_tc = raw_tc.reshape(B, T_tc)

    g, z = pl.pallas_call(
        functools.partial(_tail_kernel, nB=B, T=T),
        in_specs=[
            pl.BlockSpec((B, T_tc), lambda: (0, 0)),
            pl.BlockSpec((B, _T_SC), lambda: (0, 0)),
            pl.BlockSpec((B, T), lambda: (0, 0)),
            pl.BlockSpec((1, 1), lambda: (0, 0)),
        ],
        out_specs=[
            pl.BlockSpec((B, T), lambda: (0, 0)),
            pl.BlockSpec((B, T), lambda: (0, 0)),
        ],
        out_shape=[
            jax.ShapeDtypeStruct((B, T), jnp.float32),
            jax.ShapeDtypeStruct((B, T), jnp.float32),
        ],
    )(raw_tc, raw_sc.reshape(B, _T_SC), attn, b2)
    return g, z


# speedup vs baseline: 2.2430x; 2.2430x over previous
import functools

import jax
import jax.numpy as jnp
from jax.experimental import pallas as pl
from jax.experimental.pallas import tpu as pltpu
from jax.experimental.pallas import tpu_sc as plsc

_RHO = 0.2
_TAU = 1.0
_TBLK = 2048
_T_SC = 2048
_NW = 32
_CH = 64


def _sc_matvec(emb, w2, *, B, T, D, T_tc):
    wpb = _NW // B
    tpw = _T_SC // wpb
    nch = tpw // _CH
    mesh = plsc.VectorSubcoreMesh(core_axis_name="c", subcore_axis_name="s")

    @functools.partial(
        pl.kernel, mesh=mesh,
        out_type=jax.ShapeDtypeStruct((B * _T_SC,), jnp.float32),
        scratch_types=[
            pltpu.VMEM((_CH, D), jnp.float32),
            pltpu.VMEM((_CH, D), jnp.float32),
            pltpu.VMEM((1, D), jnp.float32),
            pltpu.VMEM((tpw,), jnp.float32),
            pltpu.SemaphoreType.DMA,
            pltpu.SemaphoreType.DMA,
        ])
    def k(emb_hbm, w_hbm, out_hbm, buf0, buf1, wv, outv, sem0, sem1):
        c = jax.lax.axis_index("c")
        s = jax.lax.axis_index("s")
        wid = s * 2 + c
        brow = wid // wpb
        tok0 = T_tc + (wid % wpb) * tpw

        def rnd(x):
            u = jax.lax.bitcast_convert_type(x, jnp.int32)
            u = (u + jnp.int32(0x7FFF) +
                 (jax.lax.shift_right_logical(u, 16) & 1)) & jnp.int32(-65536)
            return jax.lax.bitcast_convert_type(u, jnp.float32)

        pltpu.sync_copy(w_hbm, wv)

        bufs = (buf0, buf1)
        sems = (sem0, sem1)

        def start(i):
            pltpu.make_async_copy(
                emb_hbm.at[brow, pl.ds(tok0 + i * _CH, _CH), :], bufs[i % 2],
                sems[i % 2]).start()

        def wait(i):
            pltpu.make_async_copy(
                emb_hbm.at[brow, pl.ds(tok0, _CH), :], bufs[i % 2],
                sems[i % 2]).wait()

        start(0)
        start(1)
        lane = jax.lax.iota(jnp.int32, 16)
        perms = [lane ^ sh for sh in (8, 4, 2, 1)]
        wregs = [rnd(wv[0, pl.ds(j * 16, 16)]) for j in range(D // 16)]
        for i in range(nch):
            wait(i)
            buf = bufs[i % 2]

            def group(g, _):
                def row(r, res):
                    acc = jnp.zeros((16,), jnp.float32)
                    base = g * 16 + r
                    for j in range(D // 16):
                        acc = acc + rnd(buf[base, pl.ds(j * 16, 16)]) * wregs[j]
                    for p in perms:
                        acc = acc + acc.at[p].get(mode="promise_in_bounds")
                    return jnp.where(lane == r, acc, res)

                res = jax.lax.fori_loop(0, 16, row,
                                        jnp.zeros((16,), jnp.float32))
                outv[pl.ds(i * _CH + g * 16, 16)] = res
                return 0

            jax.lax.fori_loop(0, _CH // 16, group, 0)
            if i + 2 < nch:
                start(i + 2)
        pltpu.sync_copy(outv, out_hbm.at[pl.ds(wid * tpw, tpw)])

    return k(emb, w2)


def _tc_stream_kernel(emb_ref, w_ref, scores_ref):
    t = pl.program_id(1)
    emb = emb_ref[0]
    w = w_ref[...]
    scores_ref[0, :, pl.ds(t * _TBLK, _TBLK)] = jax.lax.dot_general(
        w, emb, (((1,), (1,)), ((), ())),
        preferred_element_type=jnp.float32)


def _tail_kernel(raw_tc_ref, raw_sc_ref, attn_ref, b_ref, g_ref, z_ref,
                 *, nB, T):
    attn = attn_ref[...]
    bias = b_ref[0, 0]
    raw = jnp.concatenate([raw_tc_ref[...], raw_sc_ref[...]], axis=1)
    s = attn * raw + bias
    s = jnp.where(attn == 0.0, jnp.float32(-1e9), s)
    t_eff = jnp.sum(attn, axis=1, keepdims=True)
    kf = jnp.maximum(jnp.round(jnp.float32(_RHO) * t_eff), 1.0)
    s = s / jnp.float32(_TAU)
    e = jnp.exp(s - jnp.max(s, axis=1, keepdims=True))
    z = kf * (e / jnp.sum(e, axis=1, keepdims=True))

    bits = jax.lax.bitcast_convert_type(z, jnp.int32)
    key = jnp.where(bits < 0, bits ^ jnp.int32(0x7FFFFFFF), bits)
    ub = jax.lax.bitcast_convert_type(key, jnp.uint32) ^ jnp.uint32(0x80000000)
    ki = kf.astype(jnp.int32)

    def _cnt_ge(c):
        return jnp.sum((ub >= c).astype(jnp.int32), axis=1, keepdims=True)

    def sbody(i, v):
        p_hi = (jnp.int32(15) - i) * 2 + 1
        b_hi = jax.lax.shift_left(jnp.uint32(1), p_hi.astype(jnp.uint32))
        b_lo = jax.lax.shift_left(jnp.uint32(1), (p_hi - 1).astype(jnp.uint32))
        c01 = v | b_lo
        c10 = v | b_hi
        c11 = c10 | b_lo
        n01, n10, n11 = _cnt_ge(c01), _cnt_ge(c10), _cnt_ge(c11)
        return jnp.where(n11 >= ki, c11,
                         jnp.where(n10 >= ki, c10,
                                   jnp.where(n01 >= ki, c01, v)))

    v = jax.lax.fori_loop(0, 16, sbody, jnp.zeros((nB, 1), jnp.uint32))

    gt = ub > v
    eq = ub == v
    need = ki - jnp.sum(gt.astype(jnp.int32), axis=1, keepdims=True)
    c_eq = jnp.sum(eq.astype(jnp.int32), axis=1, keepdims=True)
    idx = jax.lax.broadcasted_iota(jnp.int32, (1, T), 1)

    ties = jnp.logical_not(jnp.all(c_eq == need))

    def ibody(i, lohi):
        lo, hi = lohi
        mid = (lo + hi) // 2
        cnt = jnp.sum((eq & (idx <= mid)).astype(jnp.int32),
                      axis=1, keepdims=True)
        take = cnt >= need
        return (jnp.where(take, lo, mid + 1), jnp.where(take, mid, hi))

    lo0 = jnp.where(ties, 0, T - 1) * jnp.ones((nB, 1), jnp.int32)
    hi0 = jnp.full((nB, 1), T - 1, jnp.int32)
    trip = jnp.where(ties, 13, 0)
    lo, _ = jax.lax.fori_loop(0, trip, ibody, (lo0, hi0))
    h = (gt | (eq & (idx <= lo) & (need > 0))).astype(jnp.float32)
    g_ref[...] = h
    z_ref[...] = z


def kernel(embeddings, attn, W, b):
    B, T, D = embeddings.shape
    T_tc = T - _T_SC
    nT = T_tc // _TBLK
    b2 = b.reshape(1, 1)

    raw_sc = _sc_matvec(embeddings, W, B=B, T=T, D=D, T_tc=T_tc)

    raw_tc = pl.pallas_call(
        _tc_stream_kernel,
        grid=(B, nT),
        in_specs=[
            pl.BlockSpec((1, _TBLK, D), lambda bb, tt: (bb, tt, 0)),
            pl.BlockSpec((1, D), lambda bb, tt: (0, 0)),
        ],
        out_specs=pl.BlockSpec((1, 1, T_tc), lambda bb, tt: (bb, 0, 0)),
        out_shape=jax.ShapeDtypeStruct((B, 1, T_tc), jnp.float32),
    )(embeddings, W)
    raw_tc = raw_tc.reshape(B, T_tc)

    g, z = pl.pallas_call(
        functools.partial(_tail_kernel, nB=B, T=T),
        in_specs=[
            pl.BlockSpec((B, T_tc), lambda: (0, 0)),
            pl.BlockSpec((B, _T_SC), lambda: (0, 0)),
            pl.BlockSpec((B, T), lambda: (0, 0)),
            pl.BlockSpec((1, 1), lambda: (0, 0)),
        ],
        out_specs=[
            pl.BlockSpec((B, T), lambda: (0, 0)),
            pl.BlockSpec((B, T), lambda: (0, 0)),
        ],
        out_shape=[
            jax.ShapeDtypeStruct((B, T), jnp.float32),
            jax.ShapeDtypeStruct((B, T), jnp.float32),
        ],
    )(raw_tc, raw_sc.reshape(B, _T_SC), attn, b2)
    return g, z
